# Initial kernel scaffold; baseline (speedup 1.0000x reference)
import jax
import jax.numpy as jnp
from jax.experimental import pallas as pl


def kernel(patch_embeddings):
    raise NotImplementedError("write your pallas kernel here")


# speedup vs baseline: 1.0744x; 1.0744x over previous
import functools

import numpy as np
import jax
import jax.numpy as jnp
from jax import lax
from jax.experimental import pallas as pl
from jax.experimental.pallas import tpu as pltpu
from jax.experimental.pallas import tpu_sc as plsc

_MASKING_PERCENTAGE = 0.75

_B, _N, _D = 64, 1024, 192
_NUNM = _N - int(_MASKING_PERCENTAGE * _N)
_NC, _NS = 2, 16
_NW = _NC * _NS
_ROWS_PW = _B * _NUNM // _NW
_CHUNK = 128
_NCHUNK = _ROWS_PW // _CHUNK
_MASK_PW = (_B // _NW) * _N
_LANES = 16


def _threefry2x32(k0, k1, x0, x1):
    x0 = np.atleast_1d(np.asarray(x0, np.uint32)).copy()
    x1 = np.atleast_1d(np.asarray(x1, np.uint32)).copy()
    ks = [np.uint32(k0), np.uint32(k1),
          np.uint32(k0) ^ np.uint32(k1) ^ np.uint32(0x1BD11BDA)]
    rot = [[13, 15, 26, 6], [17, 29, 16, 24]]
    x0 += ks[0]
    x1 += ks[1]
    for i in range(5):
        for r in rot[i % 2]:
            x0 += x1
            x1 = ((x1 << np.uint32(r)) | (x1 >> np.uint32(32 - r))) ^ x0
        x0 += ks[(i + 1) % 3]
        x1 += ks[(i + 2) % 3] + np.uint32(i + 1)
    return x0, x1


@functools.lru_cache(maxsize=None)
def _mask_constants(batch, num_patches):
    n_mask = int(_MASKING_PERCENTAGE * num_patches)
    f0, f1 = _threefry2x32(0, 0, np.uint32(0), np.uint32(1))
    cnt = np.arange(batch * num_patches, dtype=np.uint64)
    o0, o1 = _threefry2x32(f0[0], f1[0],
                           (cnt >> np.uint64(32)).astype(np.uint32),
                           (cnt & np.uint64(0xFFFFFFFF)).astype(np.uint32))
    bits = o0 ^ o1
    scores = (((bits >> np.uint32(9)) | np.float32(1.0).view(np.uint32))
              .view(np.float32) - np.float32(1.0))
    scores = np.maximum(np.float32(0.0), scores).reshape(batch, num_patches)
    perm = np.argsort(scores, axis=1, kind="stable")
    masked = np.sort(perm[:, :n_mask], axis=1)
    unmasked = np.sort(perm[:, n_mask:], axis=1)
    return masked.astype(np.int32), unmasked.astype(np.int32)


_sc_mesh = plsc.VectorSubcoreMesh(
    core_axis_name="c", subcore_axis_name="s",
    num_cores=_NC, num_subcores=_NS)


@functools.partial(
    pl.kernel,
    out_type=(
        jax.ShapeDtypeStruct((_NW * _NCHUNK, _CHUNK, _D), jnp.float32),
        jax.ShapeDtypeStruct((_B * _N,), jnp.int32),
    ),
    mesh=_sc_mesh,
    scratch_types=(
        pltpu.VMEM((_NCHUNK, _CHUNK), jnp.int32),
        pltpu.VMEM((_NCHUNK, _CHUNK, _D), jnp.float32),
        pltpu.VMEM((_MASK_PW,), jnp.int32),
        pltpu.SemaphoreType.DMA,
    ),
    compiler_params=pltpu.CompilerParams(needs_layout_passes=False,
                                         use_tc_tiling_on_sc=False),
)
def _sc_gather_mask(table_hbm, idx_hbm, out_hbm, mask_hbm,
                    idx_v, rows_v, mask_v, sem):
    wid = lax.axis_index("s") * _NC + lax.axis_index("c")

    pltpu.sync_copy(idx_hbm.at[pl.ds(wid * _NCHUNK, _NCHUNK)], idx_v)

    copies = [
        pltpu.async_copy(table_hbm.at[idx_v.at[j]], rows_v.at[j], sem)
        for j in range(_NCHUNK)
    ]

    ones = jnp.ones((_LANES,), jnp.int32)
    for i in range(_MASK_PW // _LANES):
        mask_v[pl.ds(i * _LANES, _LANES)] = ones
    zeros = jnp.zeros((_LANES,), jnp.int32)
    off = wid * _MASK_PW
    for j in range(_NCHUNK):
        for k in range(_CHUNK // _LANES):
            iv = idx_v[j, pl.ds(k * _LANES, _LANES)]
            plsc.store_scatter(mask_v, [iv - off], zeros)
    pltpu.sync_copy(mask_v, mask_hbm.at[pl.ds(wid * _MASK_PW, _MASK_PW)])

    for c in copies:
        c.wait()
    pltpu.sync_copy(rows_v, out_hbm.at[pl.ds(wid * _NCHUNK, _NCHUNK)])


def kernel(patch_embeddings):
    batch, num_patches, embed_dim = patch_embeddings.shape
    masked_np, unmasked_np = _mask_constants(batch, num_patches)

    flat_idx = (np.arange(batch, dtype=np.int32)[:, None] * num_patches
                + unmasked_np).reshape(_NW * _NCHUNK, _CHUNK)

    table = patch_embeddings.reshape(batch * num_patches, embed_dim)
    patches_flat, mask_i32 = _sc_gather_mask(table, jnp.asarray(flat_idx))

    unmasked_patches = patches_flat.reshape(batch, _NUNM, embed_dim)
    bool_mask = mask_i32.reshape(batch, num_patches).astype(bool)
    return (unmasked_patches, bool_mask,
            jnp.asarray(masked_np), jnp.asarray(unmasked_np))
